# baseline (speedup 1.0000x reference)
import jax
import jax.numpy as jnp
from jax.experimental import pallas as pl
from jax.experimental.pallas import tpu as pltpu

VOCAB = 7
NAUX = 5
HID = 768


def _onehot_aux_kernel(ids_ref, aux_ref, out_ref):
    R = out_ref.shape[0]
    ids = ids_ref[:]
    col = jax.lax.broadcasted_iota(jnp.int32, (R, 128), 1)
    acc = (col == ids).astype(jnp.float32)
    for j in range(NAUX):
        acc = jnp.where(col == VOCAB + j, aux_ref[:, j : j + 1], acc)
    out_ref[:, :128] = acc
    out_ref[:, 128:] = jnp.zeros((R, HID - 128), jnp.float32)


def kernel(input_ids, aux_features):
    B, T = input_ids.shape
    N = B * T
    ids2 = input_ids.reshape(N, 1).astype(jnp.int32)
    aux2 = aux_features.reshape(N, NAUX)

    R = 1024
    grid = (N // R,)
    out = pl.pallas_call(
        _onehot_aux_kernel,
        grid=grid,
        in_specs=[
            pl.BlockSpec((R, 1), lambda i: (i, 0)),
            pl.BlockSpec((R, NAUX), lambda i: (i, 0)),
        ],
        out_specs=pl.BlockSpec((R, HID), lambda i: (i, 0)),
        out_shape=jax.ShapeDtypeStruct((N, HID), jnp.float32),
        compiler_params=pltpu.CompilerParams(
            dimension_semantics=("parallel",),
        ),
    )(ids2, aux2)
    return out.reshape(B, T, HID)


# speedup vs baseline: 1.1534x; 1.1534x over previous
import jax
import jax.numpy as jnp
from jax.experimental import pallas as pl
from jax.experimental.pallas import tpu as pltpu

VOCAB = 7
NAUX = 5
HID = 768


def _onehot_aux_kernel(ids_ref, aux_ref, out_ref):
    R = out_ref.shape[0]
    ids = ids_ref[:]
    col = jax.lax.broadcasted_iota(jnp.int32, (R, HID), 1)
    acc = (col == ids).astype(jnp.float32)
    for j in range(NAUX):
        acc = jnp.where(col == VOCAB + j, aux_ref[:, j : j + 1], acc)
    out_ref[:] = acc


def kernel(input_ids, aux_features):
    B, T = input_ids.shape
    N = B * T
    ids2 = input_ids.reshape(N, 1).astype(jnp.int32)
    aux2 = aux_features.reshape(N, NAUX)

    R = 2048
    grid = (N // R,)
    out = pl.pallas_call(
        _onehot_aux_kernel,
        grid=grid,
        in_specs=[
            pl.BlockSpec((R, 1), lambda i: (i, 0)),
            pl.BlockSpec((R, NAUX), lambda i: (i, 0)),
        ],
        out_specs=pl.BlockSpec((R, HID), lambda i: (i, 0)),
        out_shape=jax.ShapeDtypeStruct((N, HID), jnp.float32),
        compiler_params=pltpu.CompilerParams(
            dimension_semantics=("parallel",),
        ),
    )(ids2, aux2)
    return out.reshape(B, T, HID)


# speedup vs baseline: 1.1940x; 1.0352x over previous
import jax
import jax.numpy as jnp
from jax.experimental import pallas as pl
from jax.experimental.pallas import tpu as pltpu

VOCAB = 7
NAUX = 5
HID = 768


def _onehot_aux_kernel(ids_ref, aux_ref, out_ref):
    R = out_ref.shape[0]
    ids = ids_ref[:]
    col = jax.lax.broadcasted_iota(jnp.int32, (R, HID), 1)
    acc = (col == ids).astype(jnp.float32)
    for j in range(NAUX):
        acc = jnp.where(col == VOCAB + j, aux_ref[:, j : j + 1], acc)
    out_ref[:] = acc


def kernel(input_ids, aux_features):
    B, T = input_ids.shape
    N = B * T
    ids2 = input_ids.reshape(N, 1).astype(jnp.int32)
    aux2 = aux_features.reshape(N, NAUX)

    R = 4096
    grid = (N // R,)
    out = pl.pallas_call(
        _onehot_aux_kernel,
        grid=grid,
        in_specs=[
            pl.BlockSpec((R, 1), lambda i: (i, 0)),
            pl.BlockSpec((R, NAUX), lambda i: (i, 0)),
        ],
        out_specs=pl.BlockSpec((R, HID), lambda i: (i, 0)),
        out_shape=jax.ShapeDtypeStruct((N, HID), jnp.float32),
        compiler_params=pltpu.CompilerParams(
            dimension_semantics=("parallel",),
        ),
    )(ids2, aux2)
    return out.reshape(B, T, HID)
